# baseline (speedup 1.0000x reference)
import functools

import jax
import jax.numpy as jnp
from jax import lax
from jax.experimental import pallas as pl
from jax.experimental.pallas import tpu as pltpu
from jax.experimental.pallas import tpu_sc as plsc

_N = 128
_P = 64
_B = 4096
_NN = _N * _N
_EPS = 1e-06


def _table_body(s_ref, g_ref, m_ref):
    s = s_ref[...]
    g = g_ref[...]
    row = lax.broadcasted_iota(jnp.int32, (_N, _N), 0)
    col = lax.broadcasted_iota(jnp.int32, (_N, _N), 1)
    offdiag = (row != col).astype(s.dtype)
    sz = s * offdiag[None, :, :]
    denom = jnp.maximum(jnp.sum(jnp.abs(sz), axis=-1, keepdims=True), _EPS)
    sp = jnp.maximum(g, 0.0) + jnp.log1p(jnp.exp(-jnp.abs(g))) + 1e-06
    sp = sp * (_N / jnp.maximum(jnp.sum(sp, axis=-1, keepdims=True), _EPS))
    m_ref[...] = (sz / denom) * sp[:, :, None]


def _build_table(S, G):
    return pl.pallas_call(
        _table_body,
        out_shape=jax.ShapeDtypeStruct((_P, _N, _N), jnp.float32),
    )(S, G)


try:
    _info = plsc.get_sparse_core_info()
    _NC, _NS = _info.num_cores, _info.num_subcores
except ValueError:
    _NC, _NS = 2, 16
_NW = _NC * _NS
_BPW = _B // _NW
_CH = 2
_NCHUNK = _BPW // _CH


def _gather_body(table_hbm, idx_hbm, out_hbm, idx_v, rows_v, sem0, sem1):
    wid = lax.axis_index("s") * _NC + lax.axis_index("c")
    base = wid * _BPW
    cbase = wid * _NCHUNK
    pltpu.sync_copy(idx_hbm.at[pl.ds(cbase, _NCHUNK)], idx_v)

    def body(i, carry):
        c0 = i * 2
        g0 = pltpu.async_copy(table_hbm.at[idx_v.at[c0]], rows_v.at[0], sem0)
        g1 = pltpu.async_copy(table_hbm.at[idx_v.at[c0 + 1]], rows_v.at[1], sem1)
        g0.wait()
        pltpu.sync_copy(rows_v.at[0], out_hbm.at[pl.ds(base + c0 * _CH, _CH)])
        g1.wait()
        pltpu.sync_copy(rows_v.at[1], out_hbm.at[pl.ds(base + c0 * _CH + _CH, _CH)])
        return carry

    lax.fori_loop(0, _NCHUNK // 2, body, 0)


@functools.partial(jax.jit, static_argnames=())
def _gather(table, idx2):
    mesh = plsc.VectorSubcoreMesh(core_axis_name="c", subcore_axis_name="s")
    f = functools.partial(
        pl.kernel,
        mesh=mesh,
        out_type=jax.ShapeDtypeStruct((_B, _NN), jnp.float32),
        scratch_types=[
            pltpu.VMEM((_B // _CH // _NW, _CH), jnp.int32),
            pltpu.VMEM((2, _CH, _NN), jnp.float32),
            pltpu.SemaphoreType.DMA,
            pltpu.SemaphoreType.DMA,
        ],
    )(_gather_body)
    return f(table, idx2)


def kernel(phases, S, G):
    table = _build_table(S.astype(jnp.float32), G.astype(jnp.float32))
    table = table.reshape(_P, _NN)
    idx2 = phases.astype(jnp.int32).reshape(_B // _CH, _CH)
    out = _gather(table, idx2)
    return out.reshape(_B, _N, _N)
